# baseline (speedup 1.0000x reference)
import jax
import jax.numpy as jnp
from jax import lax
from jax.experimental import pallas as pl
from jax.experimental.pallas import tpu as pltpu
from jax.experimental.pallas import tpu_sc as plsc

D_MODEL = 64
WIDE = 128
SCALE = 8.0
LANES = 16
NC, NS = 2, 16
NW = NC * NS
CHUNK = 128
NBUF = 4
NOBUF = 2


def kernel(x, table):
    b, s = x.shape
    n = b * s
    per_w = n // NW
    n_chunks = per_w // CHUNK
    assert n_chunks % NBUF == 0 and n_chunks >= 2 * NBUF
    idx = x.reshape(n)
    t2 = table.reshape(table.shape[0] // 2, WIDE)

    @pl.kernel(
        out_type=jax.ShapeDtypeStruct((n // 2, WIDE), table.dtype),
        mesh=plsc.VectorSubcoreMesh(core_axis_name="c", subcore_axis_name="s"),
        scratch_types=[pltpu.VMEM((per_w,), jnp.int32)]
        + [pltpu.VMEM((CHUNK,), jnp.int32) for _ in range(NBUF)]
        + [pltpu.VMEM((CHUNK, WIDE), jnp.float32) for _ in range(NBUF)]
        + [pltpu.VMEM((CHUNK // 2, WIDE), jnp.float32) for _ in range(NOBUF)]
        + [pltpu.SemaphoreType.DMA((NBUF,)), pltpu.SemaphoreType.DMA((NOBUF,))],
    )
    def gather_rows(t2_hbm, i_hbm, o_hbm, idx_v, *bufs_and_sems):
        hidx = bufs_and_sems[0:NBUF]
        wbuf = bufs_and_sems[NBUF : 2 * NBUF]
        obuf = bufs_and_sems[2 * NBUF : 2 * NBUF + NOBUF]
        gsem = bufs_and_sems[2 * NBUF + NOBUF]
        osem = bufs_and_sems[2 * NBUF + NOBUF + 1]

        wid = lax.axis_index("s") * NC + lax.axis_index("c")
        base = wid * per_w
        pltpu.sync_copy(i_hbm.at[pl.ds(base, per_w)], idx_v)

        def start_gather(bi, ch):
            off = ch * CHUNK
            for v in range(CHUNK // LANES):
                sl = pl.ds(v * LANES, LANES)
                hidx[bi].at[sl][...] = idx_v.at[pl.ds(off + v * LANES, LANES)][...] >> 1
            pltpu.make_async_copy(
                t2_hbm.at[hidx[bi]], wbuf[bi], gsem.at[bi]
            ).start()

        def wait_gather(bi):
            pltpu.make_async_copy(
                t2_hbm.at[hidx[bi]], wbuf[bi], gsem.at[bi]
            ).wait()

        def select_half(bi, oi, ch):
            off = ch * CHUNK

            @pl.loop(0, CHUNK // 2)
            def _(q):
                r0 = 2 * q
                pv = idx_v.at[pl.ds(off + r0, 2)][...]
                h0 = (pv[0] & 1) * D_MODEL
                h1 = (pv[1] & 1) * D_MODEL
                for v in range(D_MODEL // LANES):
                    obuf[oi].at[q, pl.ds(v * LANES, LANES)][...] = (
                        wbuf[bi].at[r0, pl.ds(h0 + v * LANES, LANES)][...]
                    )
                    obuf[oi].at[q, pl.ds(D_MODEL + v * LANES, LANES)][...] = (
                        wbuf[bi].at[r0 + 1, pl.ds(h1 + v * LANES, LANES)][...]
                    )

        def start_out(oi, ch):
            row = pl.multiple_of((base + ch * CHUNK) // 2, CHUNK // 2)
            pltpu.make_async_copy(
                obuf[oi], o_hbm.at[pl.ds(row, CHUNK // 2)], osem.at[oi]
            ).start()

        def wait_out(oi):
            pltpu.make_async_copy(
                obuf[oi],
                o_hbm.at[pl.ds(pl.multiple_of(base // 2, CHUNK // 2), CHUNK // 2)],
                osem.at[oi],
            ).wait()

        for bi in range(NBUF):
            start_gather(bi, bi)

        for bi in range(NBUF):
            wait_gather(bi)
            oi = bi % NOBUF
            if bi >= NOBUF:
                wait_out(oi)
            select_half(bi, oi, bi)
            start_out(oi, bi)
            start_gather(bi, bi + NBUF)

        @pl.loop(1, n_chunks // NBUF - 1)
        def _(m):
            ch0 = m * NBUF
            for bi in range(NBUF):
                wait_gather(bi)
                oi = bi % NOBUF
                wait_out(oi)
                select_half(bi, oi, ch0 + bi)
                start_out(oi, ch0 + bi)
                start_gather(bi, ch0 + bi + NBUF)

        ch0 = n_chunks - NBUF
        for bi in range(NBUF):
            wait_gather(bi)
            oi = bi % NOBUF
            wait_out(oi)
            select_half(bi, oi, ch0 + bi)
            start_out(oi, ch0 + bi)
        for oi in range(NOBUF):
            wait_out(oi)

    out = gather_rows(t2, idx)
    return out.reshape(b, s, D_MODEL) * SCALE


# speedup vs baseline: 1.3298x; 1.3298x over previous
import jax
import jax.numpy as jnp
from jax import lax
from jax.experimental import pallas as pl
from jax.experimental.pallas import tpu as pltpu
from jax.experimental.pallas import tpu_sc as plsc

D_MODEL = 64
WIDE = 128
SCALE = 8.0
NC, NS = 2, 16
NW = NC * NS
CHUNK = 128
NBUF = 6
LAG = 3


def kernel(x, table):
    b, s = x.shape
    n = b * s
    per_w = n // NW
    n_chunks = per_w // CHUNK
    n_steady = (n_chunks - LAG - NBUF) // NBUF
    n_tail1 = (n_chunks - LAG - NBUF) % NBUF
    assert n_chunks >= 2 * NBUF
    idx = x.reshape(n)
    tpad = jnp.pad(table * SCALE, ((0, 0), (0, WIDE - D_MODEL)))

    @pl.kernel(
        out_type=jax.ShapeDtypeStruct((n, WIDE), table.dtype),
        mesh=plsc.VectorSubcoreMesh(core_axis_name="c", subcore_axis_name="s"),
        scratch_types=[pltpu.VMEM((per_w,), jnp.int32)]
        + [pltpu.VMEM((CHUNK, WIDE), jnp.float32) for _ in range(NBUF)]
        + [pltpu.SemaphoreType.DMA((NBUF,)), pltpu.SemaphoreType.DMA((NBUF,))],
    )
    def gather_rows(t_hbm, i_hbm, o_hbm, idx_v, *bufs_and_sems):
        wbuf = bufs_and_sems[0:NBUF]
        gsem = bufs_and_sems[NBUF]
        osem = bufs_and_sems[NBUF + 1]

        wid = lax.axis_index("s") * NC + lax.axis_index("c")
        base = wid * per_w
        pltpu.sync_copy(i_hbm.at[pl.ds(base, per_w)], idx_v)

        def start_gather(bi, ch):
            pltpu.make_async_copy(
                t_hbm.at[idx_v.at[pl.ds(ch * CHUNK, CHUNK)]], wbuf[bi], gsem.at[bi]
            ).start()

        def wait_gather(bi, ch):
            pltpu.make_async_copy(
                t_hbm.at[idx_v.at[pl.ds(ch * CHUNK, CHUNK)]], wbuf[bi], gsem.at[bi]
            ).wait()

        def start_out(bi, ch):
            row = pl.multiple_of(base + ch * CHUNK, CHUNK)
            pltpu.make_async_copy(
                wbuf[bi], o_hbm.at[pl.ds(row, CHUNK)], osem.at[bi]
            ).start()

        def wait_out(bi):
            pltpu.make_async_copy(
                wbuf[bi],
                o_hbm.at[pl.ds(pl.multiple_of(base, CHUNK), CHUNK)],
                osem.at[bi],
            ).wait()

        def consume(bi, ch):
            wait_gather(bi, ch)
            start_out(bi, ch)

        def refill(bj, c2, first_lap):
            if not first_lap:
                wait_out(bj)
            start_gather(bj, c2)

        for c in range(LAG):
            start_gather(c % NBUF, c)

        for ch in range(NBUF):
            c2 = ch + LAG
            refill(c2 % NBUF, c2, first_lap=c2 < NBUF)
            consume(ch % NBUF, ch)

        @pl.loop(0, n_steady)
        def _(m):
            ch0 = NBUF + m * NBUF
            for i in range(NBUF):
                refill((i + LAG) % NBUF, ch0 + i + LAG, first_lap=False)
                consume(i, ch0 + i)

        for ch in range(n_chunks - LAG - n_tail1, n_chunks - LAG):
            refill((ch + LAG) % NBUF, ch + LAG, first_lap=False)
            consume(ch % NBUF, ch)

        for ch in range(n_chunks - LAG, n_chunks):
            consume(ch % NBUF, ch)

        for bi in range(NBUF):
            wait_out(bi)

    out = gather_rows(tpad, idx)
    return out[:, :D_MODEL].reshape(b, s, D_MODEL)
